# baseline (speedup 1.0000x reference)
import jax
import jax.numpy as jnp
from jax.experimental import pallas as pl
from jax.experimental.pallas import tpu as pltpu

_TEMP_INV = 10.0
_N = 2048
_R = 8
_BI = 256
_H = 1024
_NH = _N // _H
_NS = _H // _BI


def _body(pr_ref, tr_ref, pc_ref, tc_ref, out_ref, acc_ref):
    r = pl.program_id(0)
    h = pl.program_id(1)

    @pl.when(jnp.logical_and(r == 0, h == 0))
    def _():
        acc_ref[3] = 0.0

    @pl.when(h == 0)
    def _():
        acc_ref[0] = 0.0
        acc_ref[1] = 0.0
        acc_ref[2] = 0.0

    pcb = jnp.broadcast_to(pc_ref[0, :, :], (_N, _BI))
    tcb = jnp.broadcast_to(tc_ref[0, :, :], (_N, _BI))

    ones_row = jnp.ones((1, _N), jnp.bfloat16)
    dims = (((1,), (0,)), ((), ()))

    sxy = 0.0
    sxx = 0.0
    syy = 0.0
    for s in range(_NS):
        pi = pr_ref[0, 0, s * _BI:(s + 1) * _BI].reshape(1, _BI)
        ti = tr_ref[0, 0, s * _BI:(s + 1) * _BI].reshape(1, _BI)
        bp = jnp.tanh((pi - pcb).astype(jnp.bfloat16))
        bt = jnp.tanh((ti - tcb).astype(jnp.bfloat16))
        xb = 0.5 * jax.lax.dot_general(ones_row, bp, dims,
                                       preferred_element_type=jnp.float32)
        yb = 0.5 * jax.lax.dot_general(ones_row, bt, dims,
                                       preferred_element_type=jnp.float32)
        sxy += jnp.sum(xb * yb)
        sxx += jnp.sum(xb * xb)
        syy += jnp.sum(yb * yb)

    acc_ref[0] += sxy
    acc_ref[1] += sxx
    acc_ref[2] += syy

    @pl.when(h == _NH - 1)
    def _():
        vx = jnp.sqrt(acc_ref[1] / _N + 1e-8)
        vy = jnp.sqrt(acc_ref[2] / _N + 1e-8)
        rho = (acc_ref[0] / _N) / (vx * vy + 1e-8)
        acc_ref[3] += (1.0 - rho) / _R

    @pl.when(jnp.logical_and(r == _R - 1, h == _NH - 1))
    def _():
        out_ref[0, 0] = acc_ref[3]


def kernel(preds, targets):
    ap = preds * (0.5 * _TEMP_INV)
    at = targets * (0.5 * _TEMP_INV)
    ap_row = ap.reshape(_R, 1, _N)
    at_row = at.reshape(_R, 1, _N)
    ap_col = ap.reshape(_R, _N, 1)
    at_col = at.reshape(_R, _N, 1)
    out = pl.pallas_call(
        _body,
        grid=(_R, _NH),
        in_specs=[
            pl.BlockSpec((1, 1, _H), lambda r, h: (r, 0, h)),
            pl.BlockSpec((1, 1, _H), lambda r, h: (r, 0, h)),
            pl.BlockSpec((1, _N, 1), lambda r, h: (r, 0, 0)),
            pl.BlockSpec((1, _N, 1), lambda r, h: (r, 0, 0)),
        ],
        out_specs=pl.BlockSpec(memory_space=pltpu.SMEM),
        out_shape=jax.ShapeDtypeStruct((1, 1), jnp.float32),
        scratch_shapes=[pltpu.SMEM((4,), jnp.float32)],
    )(ap_row, at_row, ap_col, at_col)
    return out[0, 0]


# speedup vs baseline: 1.1925x; 1.1925x over previous
import jax
import jax.numpy as jnp
from jax.experimental import pallas as pl
from jax.experimental.pallas import tpu as pltpu

_TEMP_INV = 10.0
_N = 2048
_R = 8
_BI = 256
_H = 1024
_NH = _N // _H
_NS = _H // _BI


def _body(pr_ref, tr_ref, pc_ref, tc_ref, out_ref, acc_ref):
    r = pl.program_id(0)
    h = pl.program_id(1)

    @pl.when(jnp.logical_and(r == 0, h == 0))
    def _():
        acc_ref[3] = 0.0

    @pl.when(h == 0)
    def _():
        acc_ref[0] = 0.0
        acc_ref[1] = 0.0
        acc_ref[2] = 0.0

    pcb = jnp.broadcast_to(pc_ref[0, 0, :].reshape(_N, 1), (_N, _BI))
    tcb = jnp.broadcast_to(tc_ref[0, 0, :].reshape(_N, 1), (_N, _BI))

    ones_row = jnp.ones((1, _N), jnp.bfloat16)
    dims = (((1,), (0,)), ((), ()))

    sxy = 0.0
    sxx = 0.0
    syy = 0.0
    for s in range(_NS):
        pi = pr_ref[0, 0, s * _BI:(s + 1) * _BI].reshape(1, _BI)
        ti = tr_ref[0, 0, s * _BI:(s + 1) * _BI].reshape(1, _BI)
        bp = jnp.tanh((pi - pcb).astype(jnp.bfloat16))
        bt = jnp.tanh((ti - tcb).astype(jnp.bfloat16))
        xb = 0.5 * jax.lax.dot_general(ones_row, bp, dims,
                                       preferred_element_type=jnp.float32)
        yb = 0.5 * jax.lax.dot_general(ones_row, bt, dims,
                                       preferred_element_type=jnp.float32)
        sxy += jnp.sum(xb * yb)
        sxx += jnp.sum(xb * xb)
        syy += jnp.sum(yb * yb)

    acc_ref[0] += sxy
    acc_ref[1] += sxx
    acc_ref[2] += syy

    @pl.when(h == _NH - 1)
    def _():
        vx = jnp.sqrt(acc_ref[1] / _N + 1e-8)
        vy = jnp.sqrt(acc_ref[2] / _N + 1e-8)
        rho = (acc_ref[0] / _N) / (vx * vy + 1e-8)
        acc_ref[3] += (1.0 - rho) / _R

    @pl.when(jnp.logical_and(r == _R - 1, h == _NH - 1))
    def _():
        out_ref[0, 0] = acc_ref[3]


def kernel(preds, targets):
    ap = preds * (0.5 * _TEMP_INV)
    at = targets * (0.5 * _TEMP_INV)
    ap_row = ap.reshape(_R, 1, _N)
    at_row = at.reshape(_R, 1, _N)
    out = pl.pallas_call(
        _body,
        grid=(_R, _NH),
        in_specs=[
            pl.BlockSpec((1, 1, _H), lambda r, h: (r, 0, h)),
            pl.BlockSpec((1, 1, _H), lambda r, h: (r, 0, h)),
            pl.BlockSpec((1, 1, _N), lambda r, h: (r, 0, 0)),
            pl.BlockSpec((1, 1, _N), lambda r, h: (r, 0, 0)),
        ],
        out_specs=pl.BlockSpec(memory_space=pltpu.SMEM),
        out_shape=jax.ShapeDtypeStruct((1, 1), jnp.float32),
        scratch_shapes=[pltpu.SMEM((4,), jnp.float32)],
    )(ap_row, at_row, ap_row, at_row)
    return out[0, 0]
